# baseline (speedup 1.0000x reference)
import functools

import jax
import jax.numpy as jnp
from jax import lax
from jax.experimental import pallas as pl
from jax.experimental.pallas import tpu as pltpu
from jax.experimental.pallas import tpu_sc as plsc

_NC = 2
_NS = 16
_NW = _NC * _NS
_LANES = 16
_BLK = 512


def _flatten_block(in_ref, out_ref):
    v = in_ref[...]
    half = _BLK // 2
    m = jnp.concatenate([v[:half], v[half:]], axis=1)
    out_ref[...] = m.reshape(out_ref.shape)


def _flatten_tc(input):
    vocab, embed_dim = input.shape
    grid = vocab // _BLK
    return pl.pallas_call(
        _flatten_block,
        grid=(grid,),
        in_specs=[pl.BlockSpec((_BLK, embed_dim), lambda i: (i, 0))],
        out_specs=pl.BlockSpec((_BLK * embed_dim,), lambda i: (i,)),
        out_shape=jax.ShapeDtypeStruct((vocab * embed_dim,), jnp.float32),
    )(input)


def _gather_body(n_per_w, embed_dim, table_hbm, idx_hbm, out_hbm,
                 idx_v, out_v, sem):
    wid = lax.axis_index("s") * _NC + lax.axis_index("c")
    base = wid * n_per_w

    pltpu.sync_copy(idx_hbm.at[pl.ds(base, n_per_w)], idx_v)

    lane_iota = lax.iota(jnp.int32, _LANES)
    period = embed_dim // _LANES

    def to_flat(g, carry):
        for c in range(period):
            i = g * period + c
            col0 = (c * _LANES) % embed_dim
            v = idx_v[pl.ds(i * _LANES, _LANES)]
            idx_v[pl.ds(i * _LANES, _LANES)] = (
                ((v >> 9) << 15) + ((v & 255) << 7) + (((v >> 8) & 1) << 6)
                + (lane_iota + col0))
        return carry

    lax.fori_loop(0, n_per_w // (_LANES * period), to_flat, 0, unroll=2)

    pltpu.async_copy(table_hbm.at[idx_v], out_v, sem).wait()

    pltpu.sync_copy(out_v, out_hbm.at[pl.ds(base, n_per_w)])


def kernel(input, index):
    vocab, embed_dim = input.shape
    batch = index.shape[0]
    n = batch * embed_dim
    n_per_w = n // _NW

    table_flat = _flatten_tc(input)
    idx_flat = index.astype(jnp.int32).reshape(n)

    mesh = plsc.VectorSubcoreMesh(core_axis_name="c", subcore_axis_name="s",
                                  num_cores=_NC, num_subcores=_NS)
    body = functools.partial(_gather_body, n_per_w, embed_dim)
    out = pl.kernel(
        body,
        out_type=jax.ShapeDtypeStruct((n,), jnp.float32),
        mesh=mesh,
        scratch_types=[
            pltpu.VMEM((n_per_w,), jnp.int32),
            pltpu.VMEM((n_per_w,), jnp.float32),
            pltpu.SemaphoreType.DMA,
        ],
    )(table_flat, idx_flat)
    return out.reshape(batch, embed_dim)


# speedup vs baseline: 1.7295x; 1.7295x over previous
import functools

import jax
import jax.numpy as jnp
from jax import lax
from jax.experimental import pallas as pl
from jax.experimental.pallas import tpu as pltpu
from jax.experimental.pallas import tpu_sc as plsc

_NC = 2
_NS = 16
_NW = _NC * _NS
_LANES = 16
_BLK = 2000


def _flatten_block(in_ref, out_ref):
    out_ref[...] = in_ref[...].reshape(out_ref.shape)


def _flatten_tc(table_2d):
    rows, width = table_2d.shape
    grid = rows // _BLK
    return pl.pallas_call(
        _flatten_block,
        grid=(grid,),
        in_specs=[pl.BlockSpec((_BLK, width), lambda i: (i, 0))],
        out_specs=pl.BlockSpec((_BLK * width,), lambda i: (i,)),
        out_shape=jax.ShapeDtypeStruct((rows * width,), jnp.float32),
    )(table_2d)


def _gather_body(n_per_w, embed_dim, table_hbm, idx_hbm, out_hbm,
                 idx_v, out_v, sem):
    wid = lax.axis_index("s") * _NC + lax.axis_index("c")
    base = wid * n_per_w

    pltpu.sync_copy(idx_hbm.at[pl.ds(base, n_per_w)], idx_v)

    lane_iota = lax.iota(jnp.int32, _LANES)
    period = embed_dim // _LANES

    def to_flat(g, carry):
        for c in range(period):
            i = g * period + c
            col0 = (c * _LANES) % embed_dim
            v = idx_v[pl.ds(i * _LANES, _LANES)]
            idx_v[pl.ds(i * _LANES, _LANES)] = (
                v * embed_dim + (lane_iota + col0))
        return carry

    lax.fori_loop(0, n_per_w // (_LANES * period), to_flat, 0, unroll=2)

    pltpu.async_copy(table_hbm.at[idx_v], out_v, sem).wait()

    pltpu.sync_copy(out_v, out_hbm.at[pl.ds(base, n_per_w)])


def kernel(input, index):
    vocab, embed_dim = input.shape
    batch = index.shape[0]
    n = batch * embed_dim
    n_per_w = n // _NW

    table_2d = input.reshape(vocab * embed_dim // 128, 128)
    table_2d = jax.lax.optimization_barrier(table_2d)
    table_flat = _flatten_tc(table_2d)
    idx_flat = index.astype(jnp.int32).reshape(n)

    mesh = plsc.VectorSubcoreMesh(core_axis_name="c", subcore_axis_name="s",
                                  num_cores=_NC, num_subcores=_NS)
    body = functools.partial(_gather_body, n_per_w, embed_dim)
    out = pl.kernel(
        body,
        out_type=jax.ShapeDtypeStruct((n,), jnp.float32),
        mesh=mesh,
        scratch_types=[
            pltpu.VMEM((n_per_w,), jnp.int32),
            pltpu.VMEM((n_per_w,), jnp.float32),
            pltpu.SemaphoreType.DMA,
        ],
    )(table_flat, idx_flat)
    return out.reshape(batch, embed_dim)


# speedup vs baseline: 2.3643x; 1.3671x over previous
import functools

import jax
import jax.numpy as jnp
from jax import lax
from jax.experimental import pallas as pl
from jax.experimental.pallas import tpu as pltpu
from jax.experimental.pallas import tpu_sc as plsc

_NC = 2
_NS = 16
_NW = _NC * _NS
_LANES = 16


def _gather_body(n_per_w, embed_dim, table_hbm, idx_hbm, out_hbm,
                 idx_v, out_v, sem):
    wid = lax.axis_index("s") * _NC + lax.axis_index("c")
    base = wid * n_per_w

    pltpu.sync_copy(idx_hbm.at[pl.ds(base, n_per_w)], idx_v)

    lane_iota = lax.iota(jnp.int32, _LANES)
    period = embed_dim // _LANES

    def to_flat(g, carry):
        for c in range(period):
            i = g * period + c
            col0 = (c * _LANES) % embed_dim
            v = idx_v[pl.ds(i * _LANES, _LANES)]
            idx_v[pl.ds(i * _LANES, _LANES)] = (
                v * embed_dim + (lane_iota + col0))
        return carry

    lax.fori_loop(0, n_per_w // (_LANES * period), to_flat, 0, unroll=2)

    pltpu.async_copy(table_hbm.at[idx_v], out_v, sem).wait()

    pltpu.sync_copy(out_v, out_hbm.at[pl.ds(base, n_per_w)])


def kernel(input, index):
    vocab, embed_dim = input.shape
    batch = index.shape[0]
    n = batch * embed_dim
    n_per_w = n // _NW

    table_flat = input.reshape(vocab * embed_dim)
    idx_flat = index.astype(jnp.int32).reshape(n)

    mesh = plsc.VectorSubcoreMesh(core_axis_name="c", subcore_axis_name="s",
                                  num_cores=_NC, num_subcores=_NS)
    body = functools.partial(_gather_body, n_per_w, embed_dim)
    out = pl.kernel(
        body,
        out_type=jax.ShapeDtypeStruct((n,), jnp.float32),
        mesh=mesh,
        scratch_types=[
            pltpu.VMEM((n_per_w,), jnp.int32),
            pltpu.VMEM((n_per_w,), jnp.float32),
            pltpu.SemaphoreType.DMA,
        ],
    )(table_flat, idx_flat)
    return out.reshape(batch, embed_dim)
